# Initial kernel scaffold; baseline (speedup 1.0000x reference)
import jax
import jax.numpy as jnp
from jax.experimental import pallas as pl


def kernel(inputs, codebook):
    raise NotImplementedError("write your pallas kernel here")


# speedup vs baseline: 1.0201x; 1.0201x over previous
import functools

import jax
import jax.numpy as jnp
from jax import lax
from jax.experimental import pallas as pl
from jax.experimental.pallas import tpu as pltpu
from jax.experimental.pallas import tpu_sc as plsc

N_TOK = 16384
DIM = 64
K_CB = 1024
CC = 0.25
BR = 512
GRID = N_TOK // BR

NC, NS = 2, 16
NW = NC * NS
B_PER_W = N_TOK // NW


def _rowsum_sq(x):
    y = x * x
    p = y[:, 0:8]
    for j in range(1, 8):
        p = p + y[:, 8 * j:8 * j + 8]
    q = p[:, 0:4] + p[:, 4:8]
    r = q[:, 0:2] + q[:, 2:4]
    return r[:, 0:1] + r[:, 1:2]


def _tc_body(x_ref, c_ref, b2_ref, idx_ref, loss_ref, ppl_ref, counts_ref, sse_ref):
    i = pl.program_id(0)
    x = x_ref[...]
    c = c_ref[...]
    a2 = _rowsum_sq(x)
    dot = lax.dot_general(x, c, (((1,), (1,)), ((), ())),
                          preferred_element_type=jnp.float32)
    d2 = a2 - 2.0 * dot + b2_ref[...]
    d2 = jnp.maximum(d2, 0.0)
    dist = jnp.sqrt(d2)

    col = lax.broadcasted_iota(jnp.int32, (BR, K_CB), 1)
    minval = jnp.min(dist, axis=1, keepdims=True)
    idx = jnp.min(jnp.where(dist == minval, col, K_CB), axis=1)
    idx_ref[...] = idx.reshape(1, 1, BR)

    d2min = jnp.min(d2, axis=1)
    cnt = jnp.sum((idx[:, None] == col).astype(jnp.float32), axis=0)

    @pl.when(i == 0)
    def _():
        counts_ref[...] = jnp.zeros_like(counts_ref)
        sse_ref[0, 0] = 0.0

    counts_ref[0, :] += cnt
    sse_ref[0, 0] += jnp.sum(d2min)

    @pl.when(i == GRID - 1)
    def _():
        loss = (1.0 + CC) * sse_ref[0, 0] / (N_TOK * DIM)
        loss_ref[...] = jnp.full((1, 1), loss, jnp.float32)
        p = counts_ref[0, :] * (1.0 / N_TOK)
        ppl = jnp.exp(-jnp.sum(p * jnp.log(p + 1e-10)))
        ppl_ref[...] = jnp.full((1, 1), ppl, jnp.float32)


_tc_call = pl.pallas_call(
    _tc_body,
    grid=(GRID,),
    in_specs=[
        pl.BlockSpec((BR, DIM), lambda i: (i, 0)),
        pl.BlockSpec((K_CB, DIM), lambda i: (0, 0)),
        pl.BlockSpec((1, K_CB), lambda i: (0, 0)),
    ],
    out_specs=[
        pl.BlockSpec((1, 1, BR), lambda i: (i, 0, 0)),
        pl.BlockSpec((1, 1), lambda i: (0, 0)),
        pl.BlockSpec((1, 1), lambda i: (0, 0)),
    ],
    out_shape=[
        jax.ShapeDtypeStruct((GRID, 1, BR), jnp.int32),
        jax.ShapeDtypeStruct((1, 1), jnp.float32),
        jax.ShapeDtypeStruct((1, 1), jnp.float32),
    ],
    scratch_shapes=[
        pltpu.VMEM((1, K_CB), jnp.float32),
        pltpu.SMEM((1, 1), jnp.float32),
    ],
)


DPAD = 128


@functools.cache
def _make_sc_gather():
    @functools.partial(
        pl.kernel,
        mesh=plsc.VectorSubcoreMesh(core_axis_name="c", subcore_axis_name="s"),
        out_type=jax.ShapeDtypeStruct((N_TOK, DPAD), jnp.float32),
        scratch_types=[
            pltpu.VMEM((B_PER_W,), jnp.int32),
            pltpu.VMEM((B_PER_W, DPAD), jnp.float32),
            pltpu.SemaphoreType.DMA,
        ],
    )
    def _sc_gather(c_hbm, idx_hbm, out_hbm, idx_v, rows_v, sem):
        wid = lax.axis_index("s") * NC + lax.axis_index("c")
        base = wid * B_PER_W
        pltpu.sync_copy(idx_hbm.at[pl.ds(base, B_PER_W)], idx_v)
        pltpu.async_copy(c_hbm.at[idx_v], rows_v, sem).wait()
        pltpu.sync_copy(rows_v, out_hbm.at[pl.ds(base, B_PER_W)])

    return _sc_gather


def kernel(inputs, codebook):
    b2 = jnp.sum(codebook * codebook, axis=1).reshape(1, K_CB)
    idx3, loss, ppl = _tc_call(inputs, codebook, b2)
    idx = idx3.reshape(N_TOK)
    c_pad = jnp.pad(codebook, ((0, 0), (0, DPAD - DIM)))
    quantized_st = _make_sc_gather()(c_pad, idx)[:, :DIM]
    return loss[0, 0], quantized_st, ppl[0, 0]
